# baseline (speedup 1.0000x reference)
import functools

import jax
import jax.numpy as jnp
from jax.experimental import pallas as pl
from jax.experimental.pallas import tpu as pltpu

_NR_EMB = 8192
_CODE_DIM = 32
_ROWS_PER_TILE = 512


def _argmin_kernel(x_ref, x2_ref, e2_ref, emb_ref, idx_ref):
    x = x_ref[...]
    emb = emb_ref[...]

    x2 = x2_ref[...]
    e2 = e2_ref[...]
    xe = jax.lax.dot_general(
        x, emb, (((1,), (0,)), ((), ())),
        preferred_element_type=jnp.float32)
    dist = (x2 - 2.0 * xe) + e2

    idx_ref[...] = jnp.argmax(-dist, axis=1)[:, None]


def _lookup_kernel(x_ref, emb_ref, idx_ref, q_ref, loss_ref, *, n_total):
    i = pl.program_id(0)
    x = x_ref[...]
    emb = emb_ref[...]
    idx = idx_ref[...]

    lanes = jax.lax.broadcasted_iota(jnp.int32, (idx.shape[0], _NR_EMB), 1)
    onehot = (lanes == idx).astype(jnp.float32)
    q = jax.lax.dot_general(
        onehot, emb, (((1,), (1,)), ((), ())),
        precision=jax.lax.Precision.HIGHEST,
        preferred_element_type=jnp.float32)
    q_ref[...] = q

    d = q - x
    partial = jnp.sum(d * d)

    @pl.when(i == 0)
    def _init():
        loss_ref[0, 0] = 0.0

    loss_ref[0, 0] += partial * (2.0 / n_total)


def kernel(inpt, emb_mtrx):
    x = inpt.reshape(-1, inpt.shape[-1])
    n_rows = x.shape[0]
    n_tiles = n_rows // _ROWS_PER_TILE

    x2 = jnp.sum(x * x, axis=1, keepdims=True)
    e2 = jnp.sum(emb_mtrx * emb_mtrx, axis=0, keepdims=True)

    idx = pl.pallas_call(
        _argmin_kernel,
        grid=(n_tiles,),
        in_specs=[
            pl.BlockSpec((_ROWS_PER_TILE, _CODE_DIM), lambda i: (i, 0)),
            pl.BlockSpec((_ROWS_PER_TILE, 1), lambda i: (i, 0)),
            pl.BlockSpec((1, _NR_EMB), lambda i: (0, 0)),
            pl.BlockSpec((_CODE_DIM, _NR_EMB), lambda i: (0, 0)),
        ],
        out_specs=pl.BlockSpec((_ROWS_PER_TILE, 1), lambda i: (i, 0)),
        out_shape=jax.ShapeDtypeStruct((n_rows, 1), jnp.int32),
    )(x, x2, e2, emb_mtrx)

    q, loss = pl.pallas_call(
        functools.partial(_lookup_kernel, n_total=float(x.size)),
        grid=(n_tiles,),
        in_specs=[
            pl.BlockSpec((_ROWS_PER_TILE, _CODE_DIM), lambda i: (i, 0)),
            pl.BlockSpec((_CODE_DIM, _NR_EMB), lambda i: (0, 0)),
            pl.BlockSpec((_ROWS_PER_TILE, 1), lambda i: (i, 0)),
        ],
        out_specs=[
            pl.BlockSpec((_ROWS_PER_TILE, _CODE_DIM), lambda i: (i, 0)),
            pl.BlockSpec(memory_space=pltpu.SMEM),
        ],
        out_shape=[
            jax.ShapeDtypeStruct((n_rows, _CODE_DIM), jnp.float32),
            jax.ShapeDtypeStruct((1, 1), jnp.float32),
        ],
    )(x, emb_mtrx, idx)

    return (q.reshape(inpt.shape), loss[0, 0])
